# Initial kernel scaffold; baseline (speedup 1.0000x reference)
import jax
import jax.numpy as jnp
from jax.experimental import pallas as pl


def kernel(x, table):
    raise NotImplementedError("write your pallas kernel here")


# speedup vs baseline: 4.7220x; 4.7220x over previous
import functools

import jax
import jax.numpy as jnp
from jax import lax
from jax.experimental import pallas as pl
from jax.experimental.pallas import tpu as pltpu
from jax.experimental.pallas import tpu_sc as plsc

D_MODEL = 128
SCALE = float(D_MODEL) ** 0.5

NC = 2
NS = 16
NW = NC * NS
G = 128


@functools.lru_cache(maxsize=None)
def _emb_kernel(B: int):
    n_per_w = B // NW
    n_groups = n_per_w // G

    mesh = plsc.VectorSubcoreMesh(core_axis_name="c", subcore_axis_name="s")

    @functools.partial(
        pl.kernel,
        mesh=mesh,
        out_type=jax.ShapeDtypeStruct((B, D_MODEL), jnp.float32),
        scratch_types=[
            pltpu.VMEM((n_groups, G), jnp.int32),
            pltpu.VMEM((G, D_MODEL), jnp.float32),
            pltpu.SemaphoreType.DMA,
        ],
    )
    def k(x_hbm, table_hbm, out_hbm, idx_v, rows_v, sem):
        wid = lax.axis_index("s") * NC + lax.axis_index("c")
        base = wid * n_per_w
        pltpu.sync_copy(x_hbm.at[wid], idx_v)

        def group(g, carry):
            pltpu.async_copy(table_hbm.at[idx_v.at[g]], rows_v, sem).wait()

            def row(r, c):
                for j in range(D_MODEL // 16):
                    sl = pl.ds(j * 16, 16)
                    rows_v[r, sl] = rows_v[r, sl] * SCALE
                return c

            lax.fori_loop(0, G, row, 0)
            pltpu.sync_copy(rows_v, out_hbm.at[pl.ds(base + g * G, G)])
            return carry

        lax.fori_loop(0, n_groups, group, 0)

    return k


def kernel(x, table):
    s0, s1 = x.shape
    B = s0 * s1
    xi = x.reshape(NW, B // (NW * G), G).astype(jnp.int32)
    out = _emb_kernel(B)(xi, table)
    return out.reshape(s0, s1, D_MODEL)
